# Initial kernel scaffold; baseline (speedup 1.0000x reference)
import jax
import jax.numpy as jnp
from jax.experimental import pallas as pl


def kernel(x, r_ij, neighbors, pairwise_mask, f_ij, W_in2f, W_f1, b_f1, W_f2, b_f2, W_out, b_out):
    raise NotImplementedError("write your pallas kernel here")


# speedup vs baseline: 12.7906x; 12.7906x over previous
import functools

import jax
import jax.numpy as jnp
from jax import lax
from jax.experimental import pallas as pl
from jax.experimental.pallas import tpu as pltpu

_LOG2 = 0.6931471805599453


def _ssp(x):
    return jnp.maximum(x, 0.0) + jnp.log1p(jnp.exp(-jnp.abs(x))) - _LOG2


def _fused_body(x_ref, f_ref, nbh_ref, mask_ref, win_ref, wf1_ref, bf1_ref,
                wf2_ref, bf2_ref, wout_ref, bout_ref, out_ref, y_scr,
                *, ta, nn, na):
    t = pl.program_id(1)

    @pl.when(t == 0)
    def _():
        y_scr[...] = jnp.dot(x_ref[0], win_ref[...],
                             preferred_element_type=jnp.float32)

    rows = ta * nn
    ng = f_ref.shape[-1]
    f = f_ref[0].reshape(rows, ng)
    h = jnp.dot(f, wf1_ref[...], preferred_element_type=jnp.float32) + bf1_ref[...]
    h = _ssp(h)
    w = jnp.dot(h, wf2_ref[...], preferred_element_type=jnp.float32) + bf2_ref[...]

    nbh = nbh_ref[0]
    onehot = (lax.broadcasted_iota(jnp.int32, (ta, nn, na), 2)
              == nbh[:, :, None]).astype(jnp.float32)
    y_g = jnp.dot(onehot.reshape(rows, na), y_scr[...],
                  preferred_element_type=jnp.float32)

    prod = (w * y_g).reshape(ta, nn, -1) * mask_ref[0][:, :, None]
    agg = jnp.sum(prod, axis=1)
    out_ref[0] = jnp.dot(agg, wout_ref[...],
                         preferred_element_type=jnp.float32) + bout_ref[...]


def kernel(x, r_ij, neighbors, pairwise_mask, f_ij, W_in2f, W_f1, b_f1,
           W_f2, b_f2, W_out, b_out):
    Nb, Na, nin = x.shape
    Nn = neighbors.shape[-1]
    ng = f_ij.shape[-1]
    nf = W_f1.shape[-1]
    nout = W_out.shape[-1]
    ta = 32
    T = Na // ta
    nbh = neighbors.astype(jnp.int32)

    out = pl.pallas_call(
        functools.partial(_fused_body, ta=ta, nn=Nn, na=Na),
        grid=(Nb, T),
        in_specs=[
            pl.BlockSpec((1, Na, nin), lambda b, t: (b, 0, 0)),
            pl.BlockSpec((1, ta, Nn, ng), lambda b, t: (b, t, 0, 0)),
            pl.BlockSpec((1, ta, Nn), lambda b, t: (b, t, 0)),
            pl.BlockSpec((1, ta, Nn), lambda b, t: (b, t, 0)),
            pl.BlockSpec((nin, nf), lambda b, t: (0, 0)),
            pl.BlockSpec((ng, nf), lambda b, t: (0, 0)),
            pl.BlockSpec((1, nf), lambda b, t: (0, 0)),
            pl.BlockSpec((nf, nf), lambda b, t: (0, 0)),
            pl.BlockSpec((1, nf), lambda b, t: (0, 0)),
            pl.BlockSpec((nf, nout), lambda b, t: (0, 0)),
            pl.BlockSpec((1, nout), lambda b, t: (0, 0)),
        ],
        out_specs=pl.BlockSpec((1, ta, nout), lambda b, t: (b, t, 0)),
        out_shape=jax.ShapeDtypeStruct((Nb, Na, nout), jnp.float32),
        scratch_shapes=[pltpu.VMEM((Na, nf), jnp.float32)],
        compiler_params=pltpu.CompilerParams(
            dimension_semantics=("arbitrary", "arbitrary"),
        ),
    )(x, f_ij, nbh, pairwise_mask, W_in2f, W_f1, b_f1.reshape(1, -1), W_f2,
      b_f2.reshape(1, -1), W_out, b_out.reshape(1, -1))
    return out
